# Initial kernel scaffold; baseline (speedup 1.0000x reference)
import functools

import jax
import jax.numpy as jnp
from jax import lax
from jax.experimental import pallas as pl
from jax.experimental.pallas import tpu as pltpu
from jax.experimental.pallas import tpu_sc as plsc

N_NODES = 50000
N_EDGES = 1600000
IN_FEATS = 1433
HID = 16
OUT = 7

LANES = 16
N_CORES = 2
N_SUBCORES = 16
N_TILES = N_CORES * N_SUBCORES
GRAN = 128
N_GRAN = N_EDGES // GRAN
CHUNK = 20
N_CHUNKS = N_GRAN // CHUNK
CHUNK_ITERS = -(-N_CHUNKS // N_TILES)
ROWS_PER_SUB = N_NODES // N_SUBCORES
ZROWS = 125

_vector_mesh = plsc.VectorSubcoreMesh(core_axis_name="c", subcore_axis_name="s")


def _sc_degrees(src2d, dst2d):
    @functools.partial(
        pl.kernel,
        out_type=jax.ShapeDtypeStruct((2, N_TILES, N_NODES), jnp.float32),
        mesh=_vector_mesh,
        scratch_types=[
            pltpu.VMEM((N_NODES,), jnp.float32),
            pltpu.VMEM((N_NODES,), jnp.float32),
            pltpu.VMEM((CHUNK, GRAN), jnp.int32),
        ],
    )
    def deg_kernel(src_hbm, dst_hbm, out_hbm, acc0, acc1, idx_v):
        cidx = lax.axis_index("c")
        sidx = lax.axis_index("s")
        wid = sidx * N_CORES + cidx

        zeros = jnp.zeros((LANES,), jnp.float32)
        ones = jnp.ones((LANES,), jnp.float32)

        @pl.loop(0, N_NODES // LANES)
        def _(i):
            acc0[pl.ds(i * LANES, LANES)] = zeros
            acc1[pl.ds(i * LANES, LANES)] = zeros

        def count(arr_hbm, acc):
            @pl.loop(0, CHUNK_ITERS)
            def _(it):
                c = wid + it * N_TILES

                @pl.when(c < N_CHUNKS)
                def _():
                    pltpu.sync_copy(arr_hbm.at[pl.ds(c * CHUNK, CHUNK)], idx_v)

                    @pl.loop(0, CHUNK)
                    def _(j):
                        for k in range(GRAN // LANES):
                            idx16 = idx_v[j, pl.ds(k * LANES, LANES)]
                            plsc.addupdate_scatter(acc, [idx16], ones)

        count(src_hbm, acc0)
        count(dst_hbm, acc1)

        pltpu.sync_copy(acc0, out_hbm.at[0, wid])
        pltpu.sync_copy(acc1, out_hbm.at[1, wid])

    return deg_kernel(src2d, dst2d)


def _sc_propagate(h, src2d, dst2d):
    @functools.partial(
        pl.kernel,
        out_type=jax.ShapeDtypeStruct((N_CORES, N_NODES, HID), jnp.float32),
        mesh=_vector_mesh,
        scratch_types=[
            pltpu.VMEM_SHARED((N_NODES, HID), jnp.float32),
            pltpu.VMEM((CHUNK, GRAN), jnp.int32),
            pltpu.VMEM((CHUNK, GRAN), jnp.int32),
            pltpu.VMEM((CHUNK, GRAN, HID), jnp.float32),
            pltpu.VMEM((ZROWS, HID), jnp.float32),
            pltpu.SemaphoreType.DMA,
            pltpu.SemaphoreType.DMA,
        ],
    )
    def prop_kernel(h_hbm, src_hbm, dst_hbm, out_hbm,
                    acc, sidx_v, didx_v, msgs, zbuf, gsem, ssem):
        cidx = lax.axis_index("c")
        sidx = lax.axis_index("s")
        wid = sidx * N_CORES + cidx

        zeros = jnp.zeros((LANES,), jnp.float32)

        @pl.loop(0, ZROWS)
        def _(i):
            zbuf[i, :] = zeros

        @pl.loop(0, ROWS_PER_SUB // ZROWS)
        def _(t):
            pltpu.sync_copy(zbuf, acc.at[pl.ds(sidx * ROWS_PER_SUB + t * ZROWS, ZROWS)])

        plsc.subcore_barrier()

        @pl.loop(0, CHUNK_ITERS)
        def _(it):
            c = wid + it * N_TILES

            @pl.when(c < N_CHUNKS)
            def _():
                pltpu.sync_copy(src_hbm.at[pl.ds(c * CHUNK, CHUNK)], sidx_v)
                pltpu.sync_copy(dst_hbm.at[pl.ds(c * CHUNK, CHUNK)], didx_v)
                gathers = [
                    pltpu.async_copy(h_hbm.at[sidx_v.at[j]], msgs.at[j], gsem)
                    for j in range(CHUNK)
                ]
                for g in gathers:
                    g.wait()
                scatters = [
                    pltpu.async_copy(msgs.at[j], acc.at[didx_v.at[j]], ssem,
                                     add=True)
                    for j in range(CHUNK)
                ]
                for s in scatters:
                    s.wait()

        plsc.subcore_barrier()
        pltpu.sync_copy(
            acc.at[pl.ds(sidx * ROWS_PER_SUB, ROWS_PER_SUB)],
            out_hbm.at[cidx, pl.ds(sidx * ROWS_PER_SUB, ROWS_PER_SUB)],
        )

    return prop_kernel(h, src2d, dst2d)


def _tc_norms(degp):
    def body(degp_ref, out_ref):
        s = jnp.sum(degp_ref[...], axis=1)
        out_ref[...] = lax.rsqrt(jnp.maximum(s, 1.0))

    return pl.pallas_call(
        body,
        out_shape=jax.ShapeDtypeStruct((2, N_NODES), jnp.float32),
    )(degp)


def _tc_matmul1(x, w1):
    rb = 512
    grid = -(-N_NODES // rb)

    def body(x_ref, w_ref, o_ref):
        o_ref[...] = jnp.dot(x_ref[...], w_ref[...],
                             preferred_element_type=jnp.float32)

    return pl.pallas_call(
        body,
        grid=(grid,),
        in_specs=[
            pl.BlockSpec((rb, IN_FEATS), lambda i: (i, 0)),
            pl.BlockSpec((IN_FEATS, HID), lambda i: (0, 0)),
        ],
        out_specs=pl.BlockSpec((rb, HID), lambda i: (i, 0)),
        out_shape=jax.ShapeDtypeStruct((N_NODES, HID), jnp.float32),
    )(x, w1)


def _tc_scale(xw, norms):
    rb = 2048
    grid = -(-N_NODES // rb)

    def body(xw_ref, n_ref, o_ref):
        o_ref[...] = xw_ref[...] * n_ref[0][:, None]

    return pl.pallas_call(
        body,
        grid=(grid,),
        in_specs=[
            pl.BlockSpec((rb, HID), lambda i: (i, 0)),
            pl.BlockSpec((2, rb), lambda i: (0, i)),
        ],
        out_specs=pl.BlockSpec((rb, HID), lambda i: (i, 0)),
        out_shape=jax.ShapeDtypeStruct((N_NODES, HID), jnp.float32),
    )(xw, norms)


def _tc_mid(partials, norms, b1):
    rb = 2048
    grid = -(-N_NODES // rb)

    def body(p_ref, n_ref, b_ref, o_ref):
        agg = (p_ref[0] + p_ref[1]) * n_ref[1][:, None] + b_ref[...]
        o_ref[...] = jnp.maximum(agg, 0.0) * n_ref[0][:, None]

    return pl.pallas_call(
        body,
        grid=(grid,),
        in_specs=[
            pl.BlockSpec((2, rb, HID), lambda i: (0, i, 0)),
            pl.BlockSpec((2, rb), lambda i: (0, i)),
            pl.BlockSpec((1, HID), lambda i: (0, 0)),
        ],
        out_specs=pl.BlockSpec((rb, HID), lambda i: (i, 0)),
        out_shape=jax.ShapeDtypeStruct((N_NODES, HID), jnp.float32),
    )(partials, norms, b1.reshape(1, HID))


def _tc_final(partials, norms, w2, b2):
    rb = 2048
    grid = -(-N_NODES // rb)

    def body(p_ref, n_ref, w_ref, b_ref, o_ref):
        agg = (p_ref[0] + p_ref[1]) * n_ref[1][:, None]
        o_ref[...] = jnp.dot(agg, w_ref[...],
                             preferred_element_type=jnp.float32) + b_ref[...]

    return pl.pallas_call(
        body,
        grid=(grid,),
        in_specs=[
            pl.BlockSpec((2, rb, HID), lambda i: (0, i, 0)),
            pl.BlockSpec((2, rb), lambda i: (0, i)),
            pl.BlockSpec((HID, OUT), lambda i: (0, 0)),
            pl.BlockSpec((1, OUT), lambda i: (0, 0)),
        ],
        out_specs=pl.BlockSpec((rb, OUT), lambda i: (i, 0)),
        out_shape=jax.ShapeDtypeStruct((N_NODES, OUT), jnp.float32),
    )(partials, norms, w2, b2.reshape(1, OUT))


def kernel(features_, edge_index, W1, b1, W2, b2):
    ei = edge_index.astype(jnp.int32)
    src2d = ei[0].reshape(N_GRAN, GRAN)
    dst2d = ei[1].reshape(N_GRAN, GRAN)

    degp = _sc_degrees(src2d, dst2d)
    norms = _tc_norms(degp)
    xw1 = _tc_matmul1(features_, W1)
    h1s = _tc_scale(xw1, norms)
    p1 = _sc_propagate(h1s, src2d, dst2d)
    y = _tc_mid(p1, norms, b1)
    p2 = _sc_propagate(y, src2d, dst2d)
    return _tc_final(p2, norms, W2, b2)


# speedup vs baseline: 17.5881x; 17.5881x over previous
import dataclasses
import functools

import jax
import jax.numpy as jnp
from jax import lax
from jax.experimental import pallas as pl
from jax.experimental.pallas import tpu as pltpu
from jax.experimental.pallas import tpu_sc as plsc

N_NODES = 50000
N_EDGES = 1600000
IN_FEATS = 1433
HID = 16
OUT = 7

LANES = 16
N_CORES = 2
N_SUBCORES = 16
N_TILES = N_CORES * N_SUBCORES
GRAN = 128
N_GRAN = N_EDGES // GRAN
CHUNK = 20
N_CHUNKS = N_GRAN // CHUNK
CHUNK_ITERS = -(-N_CHUNKS // N_TILES)
N_PAD = 50048
ROWS_PER_SUB = N_PAD // N_SUBCORES
ZROWS = 136

_vector_mesh = plsc.VectorSubcoreMesh(core_axis_name="c", subcore_axis_name="s")

_sc_params = pltpu.CompilerParams(
    needs_layout_passes=False,
    use_tc_tiling_on_sc=False,
)


def _sc_degrees(src2d, dst2d):
    @functools.partial(
        pl.kernel,
        out_type=(jax.ShapeDtypeStruct((N_TILES, 1, N_NODES), jnp.float32),
                  jax.ShapeDtypeStruct((N_TILES, 1, N_NODES), jnp.float32)),
        mesh=_vector_mesh,
        scratch_types=[
            pltpu.VMEM((N_NODES,), jnp.float32),
            pltpu.VMEM((N_NODES,), jnp.float32),
            pltpu.VMEM((CHUNK, GRAN), jnp.int32),
        ],
        compiler_params=_sc_params,
    )
    def deg_kernel(src_hbm, dst_hbm, out0_hbm, out1_hbm, acc0, acc1, idx_v):
        cidx = lax.axis_index("c")
        sidx = lax.axis_index("s")
        wid = sidx * N_CORES + cidx

        zeros = jnp.zeros((LANES,), jnp.float32)
        ones = jnp.ones((LANES,), jnp.float32)

        @pl.loop(0, N_NODES // LANES)
        def _(i):
            acc0[pl.ds(i * LANES, LANES)] = zeros
            acc1[pl.ds(i * LANES, LANES)] = zeros

        def count(arr_hbm, acc):
            @pl.loop(0, CHUNK_ITERS)
            def _(it):
                c = wid + it * N_TILES

                @pl.when(c < N_CHUNKS)
                def _():
                    pltpu.sync_copy(arr_hbm.at[c], idx_v)

                    @pl.loop(0, CHUNK)
                    def _(j):
                        for k in range(GRAN // LANES):
                            idx16 = idx_v[j, pl.ds(k * LANES, LANES)]
                            plsc.addupdate_scatter(acc, [idx16], ones)

        count(src_hbm, acc0)
        count(dst_hbm, acc1)

        pltpu.sync_copy(acc0, out0_hbm.at[wid, 0])
        pltpu.sync_copy(acc1, out1_hbm.at[wid, 0])

    return deg_kernel(src2d, dst2d)


def _sc_propagate(h, src2d, dst2d):
    @functools.partial(
        pl.kernel,
        out_type=jax.ShapeDtypeStruct((N_CORES, N_PAD, HID), jnp.float32),
        mesh=_vector_mesh,
        scratch_types=[
            pltpu.VMEM_SHARED((N_PAD, HID), jnp.float32),
            pltpu.VMEM((CHUNK, GRAN), jnp.int32),
            pltpu.VMEM((CHUNK, GRAN), jnp.int32),
            pltpu.VMEM((CHUNK, GRAN, HID), jnp.float32),
            pltpu.VMEM((ZROWS, HID), jnp.float32),
            pltpu.SemaphoreType.DMA,
            pltpu.SemaphoreType.DMA,
        ],
        compiler_params=_sc_params,
    )
    def prop_kernel(h_hbm, src_hbm, dst_hbm, out_hbm,
                    acc, sidx_v, didx_v, msgs, zbuf, gsem, ssem):
        cidx = lax.axis_index("c")
        sidx = lax.axis_index("s")
        wid = sidx * N_CORES + cidx

        zeros = jnp.zeros((LANES,), jnp.float32)

        @pl.loop(0, ZROWS)
        def _(i):
            zbuf[i, :] = zeros

        @pl.loop(0, ROWS_PER_SUB // ZROWS)
        def _(t):
            pltpu.sync_copy(zbuf, acc.at[pl.ds(sidx * ROWS_PER_SUB + t * ZROWS, ZROWS)])

        plsc.subcore_barrier()

        @pl.loop(0, CHUNK_ITERS)
        def _(it):
            c = wid + it * N_TILES

            @pl.when(c < N_CHUNKS)
            def _():
                pltpu.sync_copy(src_hbm.at[c], sidx_v)
                pltpu.sync_copy(dst_hbm.at[c], didx_v)
                gathers = [
                    pltpu.async_copy(h_hbm.at[sidx_v.at[j]], msgs.at[j], gsem)
                    for j in range(CHUNK)
                ]
                for g in gathers:
                    g.wait()
                scatters = [
                    pltpu.async_copy(msgs.at[j], acc.at[didx_v.at[j]], ssem,
                                     add=True)
                    for j in range(CHUNK)
                ]
                for s in scatters:
                    s.wait()

        plsc.subcore_barrier()
        pltpu.sync_copy(
            acc.at[pl.ds(sidx * ROWS_PER_SUB, ROWS_PER_SUB)],
            out_hbm.at[cidx, pl.ds(sidx * ROWS_PER_SUB, ROWS_PER_SUB)],
        )

    return prop_kernel(h, src2d, dst2d)


def _tc_norms(deg0, deg1):
    def body(d0_ref, d1_ref, out_ref):
        s0 = jnp.sum(d0_ref[...], axis=(0, 1))
        s1 = jnp.sum(d1_ref[...], axis=(0, 1))
        s = jnp.concatenate([s0[None, :], s1[None, :]], axis=0)
        out_ref[...] = lax.rsqrt(jnp.maximum(s, 1.0))

    return pl.pallas_call(
        body,
        out_shape=jax.ShapeDtypeStruct((2, N_NODES), jnp.float32),
    )(deg0, deg1)


def _tc_matmul1(x, w1):
    rb = 512
    grid = -(-N_NODES // rb)

    def body(x_ref, w_ref, o_ref):
        o_ref[...] = jnp.dot(x_ref[...], w_ref[...],
                             preferred_element_type=jnp.float32)

    return pl.pallas_call(
        body,
        grid=(grid,),
        in_specs=[
            pl.BlockSpec((rb, IN_FEATS), lambda i: (i, 0)),
            pl.BlockSpec((IN_FEATS, HID), lambda i: (0, 0)),
        ],
        out_specs=pl.BlockSpec((rb, HID), lambda i: (i, 0)),
        out_shape=jax.ShapeDtypeStruct((N_NODES, HID), jnp.float32),
    )(x, w1)


def _tc_scale(xw, norms):
    rb = 2048
    grid = -(-N_NODES // rb)

    def body(xw_ref, n_ref, o_ref):
        o_ref[...] = xw_ref[...] * n_ref[0][:, None]

    return pl.pallas_call(
        body,
        grid=(grid,),
        in_specs=[
            pl.BlockSpec((rb, HID), lambda i: (i, 0)),
            pl.BlockSpec((2, rb), lambda i: (0, i)),
        ],
        out_specs=pl.BlockSpec((rb, HID), lambda i: (i, 0)),
        out_shape=jax.ShapeDtypeStruct((N_NODES, HID), jnp.float32),
    )(xw, norms)


def _tc_mid(partials, norms, b1):
    rb = 2048
    grid = -(-N_NODES // rb)

    def body(p_ref, n_ref, b_ref, o_ref):
        agg = (p_ref[0] + p_ref[1]) * n_ref[1][:, None] + b_ref[...]
        o_ref[...] = jnp.maximum(agg, 0.0) * n_ref[0][:, None]

    return pl.pallas_call(
        body,
        grid=(grid,),
        in_specs=[
            pl.BlockSpec((2, rb, HID), lambda i: (0, i, 0)),
            pl.BlockSpec((2, rb), lambda i: (0, i)),
            pl.BlockSpec((1, HID), lambda i: (0, 0)),
        ],
        out_specs=pl.BlockSpec((rb, HID), lambda i: (i, 0)),
        out_shape=jax.ShapeDtypeStruct((N_NODES, HID), jnp.float32),
    )(partials, norms, b1.reshape(1, HID))


def _tc_final(partials, norms, w2, b2):
    rb = 2048
    grid = -(-N_NODES // rb)

    def body(p_ref, n_ref, w_ref, b_ref, o_ref):
        agg = (p_ref[0] + p_ref[1]) * n_ref[1][:, None]
        o_ref[...] = jnp.dot(agg, w_ref[...],
                             preferred_element_type=jnp.float32) + b_ref[...]

    return pl.pallas_call(
        body,
        grid=(grid,),
        in_specs=[
            pl.BlockSpec((2, rb, HID), lambda i: (0, i, 0)),
            pl.BlockSpec((2, rb), lambda i: (0, i)),
            pl.BlockSpec((HID, OUT), lambda i: (0, 0)),
            pl.BlockSpec((1, OUT), lambda i: (0, 0)),
        ],
        out_specs=pl.BlockSpec((rb, OUT), lambda i: (i, 0)),
        out_shape=jax.ShapeDtypeStruct((N_NODES, OUT), jnp.float32),
    )(partials, norms, w2, b2.reshape(1, OUT))


def kernel(features_, edge_index, W1, b1, W2, b2):
    ei = edge_index.astype(jnp.int32)
    src2d = ei[0].reshape(N_CHUNKS, CHUNK, GRAN)
    dst2d = ei[1].reshape(N_CHUNKS, CHUNK, GRAN)

    deg0, deg1 = _sc_degrees(src2d, dst2d)
    norms = _tc_norms(deg0, deg1)
    xw1 = _tc_matmul1(features_, W1)
    h1s = _tc_scale(xw1, norms)
    p1 = _sc_propagate(h1s, src2d, dst2d)
    y = _tc_mid(p1, norms, b1)
    p2 = _sc_propagate(y, src2d, dst2d)
    return _tc_final(p2, norms, W2, b2)


# speedup vs baseline: 31.2893x; 1.7790x over previous
import functools

import jax
import jax.numpy as jnp
from jax import lax
from jax.experimental import pallas as pl
from jax.experimental.pallas import tpu as pltpu
from jax.experimental.pallas import tpu_sc as plsc

N_NODES = 50000
N_EDGES = 1600000
IN_FEATS = 1433
HID = 16
OUT = 7

LANES = 16
N_CORES = 2
N_SUBCORES = 16
N_TILES = N_CORES * N_SUBCORES
GRAN = 128
N_GRAN = N_EDGES // GRAN
CHUNK = 10
N_CHUNKS = N_GRAN // CHUNK
CHUNK_ITERS = -(-N_CHUNKS // N_TILES)
N_PAD = 50048
ROWS_PER_SUB = N_PAD // N_SUBCORES
ZROWS = 136
PK_ROWS = N_PAD * HID // 128
DG_ROWS = N_TILES * N_PAD // 128
NB = N_PAD // 128

_vector_mesh = plsc.VectorSubcoreMesh(core_axis_name="c", subcore_axis_name="s")

_sc_params = pltpu.CompilerParams(
    needs_layout_passes=False,
    use_tc_tiling_on_sc=False,
)


def _sc_degrees(e3d):
    @functools.partial(
        pl.kernel,
        out_type=(jax.ShapeDtypeStruct((N_TILES, N_PAD), jnp.float32),
                  jax.ShapeDtypeStruct((N_TILES, N_PAD), jnp.float32)),
        mesh=_vector_mesh,
        scratch_types=[
            pltpu.VMEM((N_PAD,), jnp.float32),
            pltpu.VMEM((N_PAD,), jnp.float32),
            pltpu.VMEM((2, CHUNK, 2, GRAN), jnp.int32),
            pltpu.SemaphoreType.DMA((2,)),
        ],
        compiler_params=_sc_params,
    )
    def deg_kernel(e_hbm, out0_hbm, out1_hbm, acc0, acc1, ebuf, isem):
        cidx = lax.axis_index("c")
        sidx = lax.axis_index("s")
        wid = sidx * N_CORES + cidx

        zeros = jnp.zeros((LANES,), jnp.float32)
        ones = jnp.ones((LANES,), jnp.float32)

        @pl.loop(0, N_PAD // LANES)
        def _(i):
            acc0[pl.ds(i * LANES, LANES)] = zeros
            acc1[pl.ds(i * LANES, LANES)] = zeros

        def issue_idx(it, slot):
            c = wid + it * N_TILES

            @pl.when(c < N_CHUNKS)
            def _():
                pltpu.async_copy(e_hbm.at[pl.ds(c * CHUNK, CHUNK)],
                                 ebuf.at[slot], isem.at[slot])

        def drain_idx(it, slot):
            c = wid + it * N_TILES

            @pl.when(c < N_CHUNKS)
            def _():
                pltpu.make_async_copy(e_hbm.at[pl.ds(0, CHUNK)],
                                      ebuf.at[slot], isem.at[slot]).wait()

        def process(it, slot):
            c = wid + it * N_TILES

            @pl.when(c < N_CHUNKS)
            def _():
                @pl.loop(0, CHUNK)
                def _(j):
                    for k in range(GRAN // LANES):
                        i16s = ebuf[slot, j, 0, pl.ds(k * LANES, LANES)]
                        plsc.addupdate_scatter(acc0, [i16s], ones)
                        i16d = ebuf[slot, j, 1, pl.ds(k * LANES, LANES)]
                        plsc.addupdate_scatter(acc1, [i16d], ones)

        issue_idx(0, 0)

        @pl.loop(0, CHUNK_ITERS // 2)
        def _(h):
            it = h * 2
            drain_idx(it, 0)
            issue_idx(it + 1, 1)
            process(it, 0)
            drain_idx(it + 1, 1)
            issue_idx(it + 2, 0)
            process(it + 1, 1)

        pltpu.sync_copy(acc0, out0_hbm.at[wid])
        pltpu.sync_copy(acc1, out1_hbm.at[wid])

    return deg_kernel(e3d)


def _sc_propagate(h, e3d):
    @functools.partial(
        pl.kernel,
        out_type=jax.ShapeDtypeStruct((N_CORES, N_PAD, HID), jnp.float32),
        mesh=_vector_mesh,
        scratch_types=[
            pltpu.VMEM_SHARED((N_PAD, HID), jnp.float32),
            pltpu.VMEM((2, CHUNK, 2, GRAN), jnp.int32),
            pltpu.VMEM((2, CHUNK, GRAN, HID), jnp.float32),
            pltpu.VMEM((ZROWS, HID), jnp.float32),
            pltpu.SemaphoreType.DMA((2,)),
            pltpu.SemaphoreType.DMA((2,)),
            pltpu.SemaphoreType.DMA((2,)),
        ],
        compiler_params=_sc_params,
    )
    def prop_kernel(h_hbm, e_hbm, out_hbm, acc, ebuf, msgs, zbuf,
                    isem, gsem, ssem):
        cidx = lax.axis_index("c")
        sidx = lax.axis_index("s")
        wid = sidx * N_CORES + cidx

        zeros = jnp.zeros((LANES,), jnp.float32)

        @pl.loop(0, ZROWS)
        def _(i):
            zbuf[i, :] = zeros

        @pl.loop(0, ROWS_PER_SUB // ZROWS)
        def _(t):
            pltpu.sync_copy(zbuf, acc.at[pl.ds(sidx * ROWS_PER_SUB + t * ZROWS, ZROWS)])

        plsc.subcore_barrier()

        def issue_idx(it, slot):
            c = wid + it * N_TILES

            @pl.when(c < N_CHUNKS)
            def _():
                pltpu.async_copy(e_hbm.at[pl.ds(c * CHUNK, CHUNK)],
                                 ebuf.at[slot], isem.at[slot])

        def body(it, slot):
            c = wid + it * N_TILES
            cp = wid + (it - 1) * N_TILES

            @pl.when(c < N_CHUNKS)
            def _():
                pltpu.make_async_copy(e_hbm.at[pl.ds(0, CHUNK)],
                                      ebuf.at[slot], isem.at[slot]).wait()
                for j in range(CHUNK):
                    pltpu.async_copy(h_hbm.at[ebuf.at[slot, j, 0]],
                                     msgs.at[slot, j], gsem.at[slot])

            @pl.when((it >= 1) & (cp < N_CHUNKS))
            def _():
                for j in range(CHUNK):
                    pltpu.make_async_copy(h_hbm.at[pl.ds(0, GRAN)],
                                          msgs.at[slot ^ 1, j],
                                          ssem.at[slot ^ 1]).wait()

            issue_idx(it + 1, slot ^ 1)

            @pl.when(c < N_CHUNKS)
            def _():
                for j in range(CHUNK):
                    pltpu.make_async_copy(h_hbm.at[pl.ds(0, GRAN)],
                                          msgs.at[slot, j], gsem.at[slot]).wait()
                for j in range(CHUNK):
                    pltpu.async_copy(msgs.at[slot, j], acc.at[ebuf.at[slot, j, 1]],
                                     ssem.at[slot], add=True)

        issue_idx(0, 0)

        @pl.loop(0, CHUNK_ITERS // 2)
        def _(h2):
            it = h2 * 2
            body(it, 0)
            body(it + 1, 1)

        clast = wid + (CHUNK_ITERS - 1) * N_TILES

        @pl.when(clast < N_CHUNKS)
        def _():
            for j in range(CHUNK):
                pltpu.make_async_copy(h_hbm.at[pl.ds(0, GRAN)],
                                      msgs.at[(CHUNK_ITERS - 1) % 2, j],
                                      ssem.at[(CHUNK_ITERS - 1) % 2]).wait()

        plsc.subcore_barrier()
        pltpu.sync_copy(
            acc.at[pl.ds(sidx * ROWS_PER_SUB, ROWS_PER_SUB)],
            out_hbm.at[cidx, pl.ds(sidx * ROWS_PER_SUB, ROWS_PER_SUB)],
        )

    return prop_kernel(h, e3d)


def _tc_norms(d0, d1):
    def body(d0_ref, d1_ref, out_ref):
        n0 = lax.rsqrt(jnp.maximum(jnp.sum(d0_ref[...], axis=0), 1.0))
        n1 = lax.rsqrt(jnp.maximum(jnp.sum(d1_ref[...], axis=0), 1.0))
        z = jnp.zeros((6, N_PAD), jnp.float32)
        out_ref[...] = jnp.concatenate([n0[None, :], n1[None, :], z], axis=0)

    return pl.pallas_call(
        body,
        out_shape=jax.ShapeDtypeStruct((8, N_PAD), jnp.float32),
    )(d0, d1)


def _tc_matmul1(xt, w1):
    cb = 512
    grid = -(-N_NODES // cb)

    def body(xt_ref, w_ref, o_ref):
        xb = xt_ref[...].astype(jnp.bfloat16)
        wb = w_ref[...].astype(jnp.bfloat16)
        o_ref[...] = lax.dot_general(xb, wb, (((0,), (0,)), ((), ())),
                                     preferred_element_type=jnp.float32)

    return pl.pallas_call(
        body,
        grid=(grid,),
        in_specs=[
            pl.BlockSpec((IN_FEATS, cb), lambda i: (0, i)),
            pl.BlockSpec((IN_FEATS, HID), lambda i: (0, 0)),
        ],
        out_specs=pl.BlockSpec((cb, HID), lambda i: (i, 0)),
        out_shape=jax.ShapeDtypeStruct((N_PAD, HID), jnp.float32),
    )(xt, w1)


def _tc_scale(xw, norms):
    rb = 2048
    grid = -(-N_PAD // rb)

    def body(xw_ref, n_ref, o_ref):
        o_ref[...] = xw_ref[...] * n_ref[0][:, None]

    return pl.pallas_call(
        body,
        grid=(grid,),
        in_specs=[
            pl.BlockSpec((rb, HID), lambda i: (i, 0)),
            pl.BlockSpec((8, rb), lambda i: (0, i)),
        ],
        out_specs=pl.BlockSpec((rb, HID), lambda i: (i, 0)),
        out_shape=jax.ShapeDtypeStruct((N_PAD, HID), jnp.float32),
    )(xw, norms)


def _tc_mid(partials, norms, b1):
    rb = 2048
    grid = -(-N_PAD // rb)

    def body(p_ref, n_ref, b_ref, o_ref):
        agg = (p_ref[0] + p_ref[1]) * n_ref[1][:, None] + b_ref[...]
        o_ref[...] = jnp.maximum(agg, 0.0) * n_ref[0][:, None]

    return pl.pallas_call(
        body,
        grid=(grid,),
        in_specs=[
            pl.BlockSpec((2, rb, HID), lambda i: (0, i, 0)),
            pl.BlockSpec((8, rb), lambda i: (0, i)),
            pl.BlockSpec((1, HID), lambda i: (0, 0)),
        ],
        out_specs=pl.BlockSpec((rb, HID), lambda i: (i, 0)),
        out_shape=jax.ShapeDtypeStruct((N_PAD, HID), jnp.float32),
    )(partials, norms, b1.reshape(1, HID))


def _tc_final(partials, norms, w2, b2):
    rb = 2048
    grid = -(-N_NODES // rb)

    def body(p_ref, n_ref, w_ref, b_ref, o_ref):
        agg = (p_ref[0] + p_ref[1]) * n_ref[1][:, None]
        o_ref[...] = lax.dot_general(w_ref[...], agg, (((0,), (1,)), ((), ())),
                                     preferred_element_type=jnp.float32
                                     ) + b_ref[...].reshape(OUT, 1)

    return pl.pallas_call(
        body,
        grid=(grid,),
        in_specs=[
            pl.BlockSpec((2, rb, HID), lambda i: (0, i, 0)),
            pl.BlockSpec((8, rb), lambda i: (0, i)),
            pl.BlockSpec((HID, OUT), lambda i: (0, 0)),
            pl.BlockSpec((1, OUT), lambda i: (0, 0)),
        ],
        out_specs=pl.BlockSpec((OUT, rb), lambda i: (0, i)),
        out_shape=jax.ShapeDtypeStruct((OUT, N_NODES), jnp.float32),
    )(partials, norms, w2, b2.reshape(1, OUT))


def kernel(features_, edge_index, W1, b1, W2, b2):
    ei = edge_index.astype(jnp.int32)
    e3d = ei.reshape(2, N_GRAN, GRAN).transpose(1, 0, 2)

    deg0, deg1 = _sc_degrees(e3d)
    norms = _tc_norms(deg0, deg1)
    xw1 = _tc_matmul1(features_.T, W1)
    h1s = _tc_scale(xw1, norms)
    p1 = _sc_propagate(h1s, e3d)
    y = _tc_mid(p1, norms, b1)
    p2 = _sc_propagate(y, e3d)
    return _tc_final(p2, norms, W2, b2).T


# speedup vs baseline: 41.8867x; 1.3387x over previous
import functools

import jax
import jax.numpy as jnp
from jax import lax
from jax.experimental import pallas as pl
from jax.experimental.pallas import tpu as pltpu
from jax.experimental.pallas import tpu_sc as plsc

N_NODES = 50000
N_EDGES = 1600000
IN_FEATS = 1433
HID = 16
OUT = 7

LANES = 16
N_CORES = 2
N_SUBCORES = 16
N_TILES = N_CORES * N_SUBCORES
GRAN = 128
N_GRAN = N_EDGES // GRAN
CHUNK = 10
N_CHUNKS = N_GRAN // CHUNK
CHUNK_ITERS = -(-N_CHUNKS // N_TILES)
N_PAD = 50048
ROWS_PER_SUB = N_PAD // N_SUBCORES
ZROWS = 136
PK_ROWS = N_PAD * HID // 128
DG_ROWS = N_TILES * N_PAD // 128
NB = N_PAD // 128

_vector_mesh = plsc.VectorSubcoreMesh(core_axis_name="c", subcore_axis_name="s")

_sc_params = pltpu.CompilerParams(
    needs_layout_passes=False,
    use_tc_tiling_on_sc=False,
)


def _sc_degrees(e3d):
    @functools.partial(
        pl.kernel,
        out_type=(jax.ShapeDtypeStruct((N_TILES, N_PAD), jnp.float32),
                  jax.ShapeDtypeStruct((N_TILES, N_PAD), jnp.float32)),
        mesh=_vector_mesh,
        scratch_types=[
            pltpu.VMEM((N_PAD,), jnp.float32),
            pltpu.VMEM((N_PAD,), jnp.float32),
            pltpu.VMEM((2, CHUNK, 2, GRAN), jnp.int32),
            pltpu.SemaphoreType.DMA((2,)),
        ],
        compiler_params=_sc_params,
    )
    def deg_kernel(e_hbm, out0_hbm, out1_hbm, acc0, acc1, ebuf, isem):
        cidx = lax.axis_index("c")
        sidx = lax.axis_index("s")
        wid = sidx * N_CORES + cidx

        zeros = jnp.zeros((LANES,), jnp.float32)
        ones = jnp.ones((LANES,), jnp.float32)

        @pl.loop(0, N_PAD // LANES)
        def _(i):
            acc0[pl.ds(i * LANES, LANES)] = zeros
            acc1[pl.ds(i * LANES, LANES)] = zeros

        def issue_idx(it, slot):
            c = wid + it * N_TILES

            @pl.when(c < N_CHUNKS)
            def _():
                pltpu.async_copy(e_hbm.at[pl.ds(c * CHUNK, CHUNK)],
                                 ebuf.at[slot], isem.at[slot])

        def drain_idx(it, slot):
            c = wid + it * N_TILES

            @pl.when(c < N_CHUNKS)
            def _():
                pltpu.make_async_copy(e_hbm.at[pl.ds(0, CHUNK)],
                                      ebuf.at[slot], isem.at[slot]).wait()

        def process(it, slot):
            c = wid + it * N_TILES

            @pl.when(c < N_CHUNKS)
            def _():
                @pl.loop(0, CHUNK)
                def _(j):
                    for k in range(GRAN // LANES):
                        i16s = ebuf[slot, j, 0, pl.ds(k * LANES, LANES)]
                        plsc.addupdate_scatter(acc0, [i16s], ones)
                        i16d = ebuf[slot, j, 1, pl.ds(k * LANES, LANES)]
                        plsc.addupdate_scatter(acc1, [i16d], ones)

        issue_idx(0, 0)

        @pl.loop(0, CHUNK_ITERS // 2)
        def _(h):
            it = h * 2
            drain_idx(it, 0)
            issue_idx(it + 1, 1)
            process(it, 0)
            drain_idx(it + 1, 1)
            issue_idx(it + 2, 0)
            process(it + 1, 1)

        pltpu.sync_copy(acc0, out0_hbm.at[wid])
        pltpu.sync_copy(acc1, out1_hbm.at[wid])

    return deg_kernel(e3d)


def _sc_propagate(h, e3d):
    @functools.partial(
        pl.kernel,
        out_type=jax.ShapeDtypeStruct((N_CORES, N_PAD, HID), jnp.float32),
        mesh=_vector_mesh,
        scratch_types=[
            pltpu.VMEM_SHARED((N_PAD, HID), jnp.float32),
            pltpu.VMEM((2, CHUNK, 2, GRAN), jnp.int32),
            pltpu.VMEM((2, CHUNK, GRAN, HID), jnp.float32),
            pltpu.VMEM((ZROWS, HID), jnp.float32),
            pltpu.SemaphoreType.DMA((2,)),
            pltpu.SemaphoreType.DMA((2,)),
            pltpu.SemaphoreType.DMA((2,)),
        ],
        compiler_params=_sc_params,
    )
    def prop_kernel(h_hbm, e_hbm, out_hbm, acc, ebuf, msgs, zbuf,
                    isem, gsem, ssem):
        cidx = lax.axis_index("c")
        sidx = lax.axis_index("s")
        wid = sidx * N_CORES + cidx

        zeros = jnp.zeros((LANES,), jnp.float32)

        @pl.loop(0, ZROWS)
        def _(i):
            zbuf[i, :] = zeros

        @pl.loop(0, ROWS_PER_SUB // ZROWS)
        def _(t):
            pltpu.sync_copy(zbuf, acc.at[pl.ds(sidx * ROWS_PER_SUB + t * ZROWS, ZROWS)])

        plsc.subcore_barrier()

        def issue_idx(it, slot):
            c = wid + it * N_TILES

            @pl.when(c < N_CHUNKS)
            def _():
                pltpu.async_copy(e_hbm.at[pl.ds(c * CHUNK, CHUNK)],
                                 ebuf.at[slot], isem.at[slot])

        def body(it, slot):
            c = wid + it * N_TILES
            cp = wid + (it - 1) * N_TILES

            @pl.when(c < N_CHUNKS)
            def _():
                pltpu.make_async_copy(e_hbm.at[pl.ds(0, CHUNK)],
                                      ebuf.at[slot], isem.at[slot]).wait()
                for j in range(CHUNK):
                    pltpu.async_copy(h_hbm.at[ebuf.at[slot, j, 0]],
                                     msgs.at[slot, j], gsem.at[slot])

            @pl.when((it >= 1) & (cp < N_CHUNKS))
            def _():
                for j in range(CHUNK):
                    pltpu.make_async_copy(h_hbm.at[pl.ds(0, GRAN)],
                                          msgs.at[slot ^ 1, j],
                                          ssem.at[slot ^ 1]).wait()

            issue_idx(it + 1, slot ^ 1)

            @pl.when(c < N_CHUNKS)
            def _():
                for j in range(CHUNK):
                    pltpu.make_async_copy(h_hbm.at[pl.ds(0, GRAN)],
                                          msgs.at[slot, j], gsem.at[slot]).wait()
                for j in range(CHUNK):
                    pltpu.async_copy(msgs.at[slot, j], acc.at[ebuf.at[slot, j, 1]],
                                     ssem.at[slot], add=True)

        issue_idx(0, 0)

        @pl.loop(0, CHUNK_ITERS // 2)
        def _(h2):
            it = h2 * 2
            body(it, 0)
            body(it + 1, 1)

        clast = wid + (CHUNK_ITERS - 1) * N_TILES

        @pl.when(clast < N_CHUNKS)
        def _():
            for j in range(CHUNK):
                pltpu.make_async_copy(h_hbm.at[pl.ds(0, GRAN)],
                                      msgs.at[(CHUNK_ITERS - 1) % 2, j],
                                      ssem.at[(CHUNK_ITERS - 1) % 2]).wait()

        plsc.subcore_barrier()
        pltpu.sync_copy(
            acc.at[pl.ds(sidx * ROWS_PER_SUB, ROWS_PER_SUB)],
            out_hbm.at[cidx, pl.ds(sidx * ROWS_PER_SUB, ROWS_PER_SUB)],
        )

    return prop_kernel(h, e3d)


def _tc_norms(d0, d1, kexp):
    def expand(n, k):
        e = lax.dot_general(n, k, (((1,), (0,)), ((), ())),
                            preferred_element_type=jnp.float32)
        return e.reshape(NB, LANES, 128).reshape(PK_ROWS, 128)

    def body(d0_ref, d1_ref, k_ref, ne0_ref, ne1_ref):
        s0 = jnp.sum(d0_ref[...].reshape(N_TILES, NB, 128), axis=0)
        s1 = jnp.sum(d1_ref[...].reshape(N_TILES, NB, 128), axis=0)
        k = k_ref[...]
        ne0_ref[...] = expand(lax.rsqrt(jnp.maximum(s0, 1.0)), k)
        ne1_ref[...] = expand(lax.rsqrt(jnp.maximum(s1, 1.0)), k)

    return pl.pallas_call(
        body,
        out_shape=(jax.ShapeDtypeStruct((PK_ROWS, 128), jnp.float32),
                   jax.ShapeDtypeStruct((PK_ROWS, 128), jnp.float32)),
    )(d0, d1, kexp)


def _tc_matmul1(xt, w1):
    cb = 512
    grid = -(-N_NODES // cb)

    def body(xt_ref, w_ref, o_ref):
        xb = xt_ref[...].astype(jnp.bfloat16)
        wb = w_ref[...].astype(jnp.bfloat16)
        prod = lax.dot_general(xb, wb, (((0,), (0,)), ((), ())),
                               preferred_element_type=jnp.float32)
        pr = prod.reshape(cb // 8, 8, HID)
        o_ref[...] = jnp.concatenate([pr[:, m, :] for m in range(8)], axis=1)

    return pl.pallas_call(
        body,
        grid=(grid,),
        in_specs=[
            pl.BlockSpec((IN_FEATS, cb), lambda i: (0, i)),
            pl.BlockSpec((IN_FEATS, HID), lambda i: (0, 0)),
        ],
        out_specs=pl.BlockSpec((cb // 8, 128), lambda i: (i, 0)),
        out_shape=jax.ShapeDtypeStruct((PK_ROWS, 128), jnp.float32),
    )(xt, w1)


def _tc_scale(xwp, ne0):
    def body(xw_ref, n_ref, o_ref):
        o_ref[...] = xw_ref[...] * n_ref[...]

    return pl.pallas_call(
        body,
        out_shape=jax.ShapeDtypeStruct((PK_ROWS, 128), jnp.float32),
    )(xwp, ne0)


def _tc_mid(p, ne0, ne1, b1e):
    def body(p_ref, n0_ref, n1_ref, b_ref, o_ref):
        pv = p_ref[...].reshape(2, PK_ROWS, 128)
        agg = (pv[0] + pv[1]) * n1_ref[...] + b_ref[...]
        o_ref[...] = jnp.maximum(agg, 0.0) * n0_ref[...]

    return pl.pallas_call(
        body,
        out_shape=jax.ShapeDtypeStruct((PK_ROWS, 128), jnp.float32),
    )(p, ne0, ne1, b1e)


def _tc_final(p, ne1, w2e, b2e):
    def body(p_ref, n_ref, w_ref, b_ref, o_ref):
        pv = p_ref[...].reshape(2, PK_ROWS, 128)
        agg = (pv[0] + pv[1]) * n_ref[...]
        o_ref[...] = jnp.dot(agg, w_ref[...],
                             preferred_element_type=jnp.float32) + b_ref[...]

    return pl.pallas_call(
        body,
        out_shape=jax.ShapeDtypeStruct((PK_ROWS, 8 * OUT), jnp.float32),
    )(p, ne1, w2e, b2e)


def kernel(features_, edge_index, W1, b1, W2, b2):
    ei = edge_index.astype(jnp.int32)
    e3d = ei.reshape(2, N_GRAN, GRAN).transpose(1, 0, 2)

    b1e = jnp.tile(b1, 8).reshape(1, 128)
    w2e = jnp.kron(jnp.eye(8, dtype=jnp.float32), W2)
    b2e = jnp.tile(b2, 8).reshape(1, 8 * OUT)
    ll = jnp.arange(LANES * 128)[None, :]
    kexp = (jnp.arange(128)[:, None] == 8 * (ll // 128) + (ll % 128) // LANES
            ).astype(jnp.float32)

    deg0, deg1 = _sc_degrees(e3d)
    ne0, ne1 = _tc_norms(deg0.reshape(DG_ROWS, 128),
                         deg1.reshape(DG_ROWS, 128), kexp)
    xw1p = _tc_matmul1(features_.T, W1)
    h1sp = _tc_scale(xw1p, ne0)
    p1 = _sc_propagate(h1sp.reshape(N_PAD, HID), e3d)
    yp = _tc_mid(p1.reshape(DG_ROWS, 128), ne0, ne1, b1e)
    p2 = _sc_propagate(yp.reshape(N_PAD, HID), e3d)
    outp = _tc_final(p2.reshape(DG_ROWS, 128), ne1, w2e, b2e)
    return outp.reshape(N_PAD, OUT)[:N_NODES]


# speedup vs baseline: 45.8076x; 1.0936x over previous
import functools

import jax
import jax.numpy as jnp
from jax import lax
from jax.experimental import pallas as pl
from jax.experimental.pallas import tpu as pltpu
from jax.experimental.pallas import tpu_sc as plsc

N_NODES = 50000
N_EDGES = 1600000
IN_FEATS = 1433
HID = 16
OUT = 7

LANES = 16
N_CORES = 2
N_SUBCORES = 16
N_TILES = N_CORES * N_SUBCORES
GRAN = 128
N_GRAN = N_EDGES // GRAN
CHUNK = 10
N_CHUNKS = N_GRAN // CHUNK
CHUNK_ITERS = -(-N_CHUNKS // N_TILES)
N_PAD = 50048
ROWS_PER_SUB = N_PAD // N_SUBCORES
ZROWS = 136
PK_ROWS = N_PAD * HID // 128
DG_ROWS = N_TILES * N_PAD // 128
NB = N_PAD // 128

_vector_mesh = plsc.VectorSubcoreMesh(core_axis_name="c", subcore_axis_name="s")

_sc_params = pltpu.CompilerParams(
    needs_layout_passes=False,
    use_tc_tiling_on_sc=False,
)


def _sc_degrees(e3d):
    @functools.partial(
        pl.kernel,
        out_type=(jax.ShapeDtypeStruct((N_TILES, N_PAD), jnp.float32),
                  jax.ShapeDtypeStruct((N_TILES, N_PAD), jnp.float32)),
        mesh=_vector_mesh,
        scratch_types=[
            pltpu.VMEM((N_PAD,), jnp.float32),
            pltpu.VMEM((N_PAD,), jnp.float32),
            pltpu.VMEM((2, CHUNK, 2, GRAN), jnp.int32),
            pltpu.SemaphoreType.DMA((2,)),
        ],
        compiler_params=_sc_params,
    )
    def deg_kernel(e_hbm, out0_hbm, out1_hbm, acc0, acc1, ebuf, isem):
        cidx = lax.axis_index("c")
        sidx = lax.axis_index("s")
        wid = sidx * N_CORES + cidx

        zeros = jnp.zeros((LANES,), jnp.float32)
        ones = jnp.ones((LANES,), jnp.float32)

        @pl.loop(0, N_PAD // LANES)
        def _(i):
            acc0[pl.ds(i * LANES, LANES)] = zeros
            acc1[pl.ds(i * LANES, LANES)] = zeros

        def issue_idx(it, slot):
            c = wid + it * N_TILES

            @pl.when(c < N_CHUNKS)
            def _():
                pltpu.async_copy(e_hbm.at[pl.ds(c * CHUNK, CHUNK)],
                                 ebuf.at[slot], isem.at[slot])

        def drain_idx(it, slot):
            c = wid + it * N_TILES

            @pl.when(c < N_CHUNKS)
            def _():
                pltpu.make_async_copy(e_hbm.at[pl.ds(0, CHUNK)],
                                      ebuf.at[slot], isem.at[slot]).wait()

        def process(it, slot):
            c = wid + it * N_TILES

            @pl.when(c < N_CHUNKS)
            def _():
                @pl.loop(0, CHUNK)
                def _(j):
                    for k in range(GRAN // LANES):
                        i16s = ebuf[slot, j, 0, pl.ds(k * LANES, LANES)]
                        plsc.addupdate_scatter(acc0, [i16s], ones)
                        i16d = ebuf[slot, j, 1, pl.ds(k * LANES, LANES)]
                        plsc.addupdate_scatter(acc1, [i16d], ones)

        issue_idx(0, 0)

        @pl.loop(0, CHUNK_ITERS // 2)
        def _(h):
            it = h * 2
            drain_idx(it, 0)
            issue_idx(it + 1, 1)
            process(it, 0)
            drain_idx(it + 1, 1)
            issue_idx(it + 2, 0)
            process(it + 1, 1)

        pltpu.sync_copy(acc0, out0_hbm.at[wid])
        pltpu.sync_copy(acc1, out1_hbm.at[wid])

    return deg_kernel(e3d)


def _sc_propagate(h, e3d):
    @functools.partial(
        pl.kernel,
        out_type=jax.ShapeDtypeStruct((N_CORES, N_PAD, HID), jnp.float32),
        mesh=_vector_mesh,
        scratch_types=[
            pltpu.VMEM_SHARED((N_PAD, HID), jnp.float32),
            pltpu.VMEM((2, CHUNK, 2, GRAN), jnp.int32),
            pltpu.VMEM((2, CHUNK, GRAN, HID), jnp.float32),
            pltpu.VMEM((ZROWS, HID), jnp.float32),
            pltpu.SemaphoreType.DMA((2,)),
            pltpu.SemaphoreType.DMA((2,)),
            pltpu.SemaphoreType.DMA((2,)),
        ],
        compiler_params=_sc_params,
    )
    def prop_kernel(h_hbm, e_hbm, out_hbm, acc, ebuf, msgs, zbuf,
                    isem, gsem, ssem):
        cidx = lax.axis_index("c")
        sidx = lax.axis_index("s")
        wid = sidx * N_CORES + cidx

        zeros = jnp.zeros((LANES,), jnp.float32)

        @pl.loop(0, ZROWS)
        def _(i):
            zbuf[i, :] = zeros

        @pl.loop(0, ROWS_PER_SUB // ZROWS)
        def _(t):
            pltpu.sync_copy(zbuf, acc.at[pl.ds(sidx * ROWS_PER_SUB + t * ZROWS, ZROWS)])

        plsc.subcore_barrier()

        def issue_idx(it, slot):
            c = wid + it * N_TILES

            @pl.when(c < N_CHUNKS)
            def _():
                pltpu.async_copy(e_hbm.at[pl.ds(c * CHUNK, CHUNK)],
                                 ebuf.at[slot], isem.at[slot])

        def body(it, slot):
            c = wid + it * N_TILES
            cp = wid + (it - 1) * N_TILES

            @pl.when(c < N_CHUNKS)
            def _():
                pltpu.make_async_copy(e_hbm.at[pl.ds(0, CHUNK)],
                                      ebuf.at[slot], isem.at[slot]).wait()
                for j in range(CHUNK):
                    pltpu.async_copy(h_hbm.at[ebuf.at[slot, j, 0]],
                                     msgs.at[slot, j], gsem.at[slot])

            @pl.when((it >= 1) & (cp < N_CHUNKS))
            def _():
                for j in range(CHUNK):
                    pltpu.make_async_copy(h_hbm.at[pl.ds(0, GRAN)],
                                          msgs.at[slot ^ 1, j],
                                          ssem.at[slot ^ 1]).wait()

            issue_idx(it + 1, slot ^ 1)

            @pl.when(c < N_CHUNKS)
            def _():
                for j in range(CHUNK):
                    pltpu.make_async_copy(h_hbm.at[pl.ds(0, GRAN)],
                                          msgs.at[slot, j], gsem.at[slot]).wait()
                for j in range(CHUNK):
                    pltpu.async_copy(msgs.at[slot, j], acc.at[ebuf.at[slot, j, 1]],
                                     ssem.at[slot], add=True)

        issue_idx(0, 0)

        @pl.loop(0, CHUNK_ITERS // 2)
        def _(h2):
            it = h2 * 2
            body(it, 0)
            body(it + 1, 1)

        clast = wid + (CHUNK_ITERS - 1) * N_TILES

        @pl.when(clast < N_CHUNKS)
        def _():
            for j in range(CHUNK):
                pltpu.make_async_copy(h_hbm.at[pl.ds(0, GRAN)],
                                      msgs.at[(CHUNK_ITERS - 1) % 2, j],
                                      ssem.at[(CHUNK_ITERS - 1) % 2]).wait()

        plsc.subcore_barrier()
        pltpu.sync_copy(
            acc.at[pl.ds(sidx * ROWS_PER_SUB, ROWS_PER_SUB)],
            out_hbm.at[cidx, pl.ds(sidx * ROWS_PER_SUB, ROWS_PER_SUB)],
        )

    return prop_kernel(h, e3d)


def _tc_norms(d0, d1, kexp):
    def expand(n, k):
        e = lax.dot_general(n, k, (((1,), (0,)), ((), ())),
                            preferred_element_type=jnp.float32)
        return e.reshape(NB, LANES, 128).reshape(PK_ROWS, 128)

    def body(d0_ref, d1_ref, k_ref, ne0_ref, ne1_ref):
        s0 = jnp.sum(d0_ref[...].reshape(N_TILES, NB, 128), axis=0)
        s1 = jnp.sum(d1_ref[...].reshape(N_TILES, NB, 128), axis=0)
        k = k_ref[...]
        ne0_ref[...] = expand(lax.rsqrt(jnp.maximum(s0, 1.0)), k)
        ne1_ref[...] = expand(lax.rsqrt(jnp.maximum(s1, 1.0)), k)

    return pl.pallas_call(
        body,
        out_shape=(jax.ShapeDtypeStruct((PK_ROWS, 128), jnp.float32),
                   jax.ShapeDtypeStruct((PK_ROWS, 128), jnp.float32)),
    )(d0, d1, kexp)


def _tc_matmul1(xt, w1):
    cb = 1024
    grid = -(-N_NODES // cb)

    def body(xt_ref, w_ref, o_ref):
        xb = xt_ref[...].astype(jnp.bfloat16)
        wb = w_ref[...].astype(jnp.bfloat16)
        prod = lax.dot_general(xb, wb, (((0,), (0,)), ((), ())),
                               preferred_element_type=jnp.float32)
        pr = prod.reshape(cb // 8, 8, HID)
        o_ref[...] = jnp.concatenate([pr[:, m, :] for m in range(8)], axis=1)

    return pl.pallas_call(
        body,
        grid=(grid,),
        in_specs=[
            pl.BlockSpec((IN_FEATS, cb), lambda i: (0, i)),
            pl.BlockSpec((IN_FEATS, HID), lambda i: (0, 0)),
        ],
        out_specs=pl.BlockSpec((cb // 8, 128), lambda i: (i, 0)),
        out_shape=jax.ShapeDtypeStruct((PK_ROWS, 128), jnp.float32),
    )(xt, w1)


def _tc_scale(xwp, ne0):
    def body(xw_ref, n_ref, o_ref):
        o_ref[...] = xw_ref[...] * n_ref[...]

    return pl.pallas_call(
        body,
        out_shape=jax.ShapeDtypeStruct((PK_ROWS, 128), jnp.float32),
    )(xwp, ne0)


def _tc_mid(p, ne0, ne1, b1e):
    def body(p_ref, n0_ref, n1_ref, b_ref, o_ref):
        pv = p_ref[...].reshape(2, PK_ROWS, 128)
        agg = (pv[0] + pv[1]) * n1_ref[...] + b_ref[...]
        o_ref[...] = jnp.maximum(agg, 0.0) * n0_ref[...]

    return pl.pallas_call(
        body,
        out_shape=jax.ShapeDtypeStruct((PK_ROWS, 128), jnp.float32),
    )(p, ne0, ne1, b1e)


def _tc_final(p, ne1, w2, b2):
    rb = 368
    grid = PK_ROWS // rb

    def body(p0_ref, p1_ref, n_ref, w_ref, b_ref, o_ref):
        agg = (p0_ref[...] + p1_ref[...]) * n_ref[...]
        un = jnp.stack([agg[:, m * HID:(m + 1) * HID] for m in range(8)],
                       axis=1).reshape(8 * rb, HID)
        res = lax.dot_general(w_ref[...], un, (((0,), (1,)), ((), ())),
                              preferred_element_type=jnp.float32)
        o_ref[...] = res + b_ref[...].reshape(OUT, 1)

    return pl.pallas_call(
        body,
        grid=(grid,),
        in_specs=[
            pl.BlockSpec((rb, 128), lambda i: (i, 0)),
            pl.BlockSpec((rb, 128), lambda i: (i + PK_ROWS // rb, 0)),
            pl.BlockSpec((rb, 128), lambda i: (i, 0)),
            pl.BlockSpec((HID, OUT), lambda i: (0, 0)),
            pl.BlockSpec((1, OUT), lambda i: (0, 0)),
        ],
        out_specs=pl.BlockSpec((OUT, 8 * rb), lambda i: (0, i)),
        out_shape=jax.ShapeDtypeStruct((OUT, N_NODES), jnp.float32),
    )(p, p, ne1, w2, b2.reshape(1, OUT))


def kernel(features_, edge_index, W1, b1, W2, b2):
    ei = edge_index.astype(jnp.int32)
    e3d = ei.reshape(2, N_GRAN, GRAN).transpose(1, 0, 2)

    b1e = jnp.tile(b1, 8).reshape(1, 128)
    ll = jnp.arange(LANES * 128)[None, :]
    kexp = (jnp.arange(128)[:, None] == 8 * (ll // 128) + (ll % 128) // LANES
            ).astype(jnp.float32)

    deg0, deg1 = _sc_degrees(e3d)
    ne0, ne1 = _tc_norms(deg0.reshape(DG_ROWS, 128),
                         deg1.reshape(DG_ROWS, 128), kexp)
    xw1p = _tc_matmul1(features_.T, W1)
    h1sp = _tc_scale(xw1p, ne0)
    p1 = _sc_propagate(h1sp.reshape(N_PAD, HID), e3d)
    yp = _tc_mid(p1.reshape(DG_ROWS, 128), ne0, ne1, b1e)
    p2 = _sc_propagate(yp.reshape(N_PAD, HID), e3d)
    return _tc_final(p2.reshape(DG_ROWS, 128), ne1, W2, b2).T


# speedup vs baseline: 47.1395x; 1.0291x over previous
import functools

import jax
import jax.numpy as jnp
from jax import lax
from jax.experimental import pallas as pl
from jax.experimental.pallas import tpu as pltpu
from jax.experimental.pallas import tpu_sc as plsc

N_NODES = 50000
N_EDGES = 1600000
IN_FEATS = 1433
HID = 16
OUT = 7

LANES = 16
N_CORES = 2
N_SUBCORES = 16
N_TILES = N_CORES * N_SUBCORES
GRAN = 128
N_GRAN = N_EDGES // GRAN
CHUNK = 10
N_CHUNKS = N_GRAN // CHUNK
CHUNK_ITERS = -(-N_CHUNKS // N_TILES)
N_PAD = 50048
ROWS_PER_SUB = N_PAD // N_SUBCORES
ZROWS = 136
PK_ROWS = N_PAD * HID // 128
DG_ROWS = N_TILES * N_PAD // 128
NB = N_PAD // 128

_vector_mesh = plsc.VectorSubcoreMesh(core_axis_name="c", subcore_axis_name="s")

_sc_params = pltpu.CompilerParams(
    needs_layout_passes=False,
    use_tc_tiling_on_sc=False,
)


def _sc_degrees(e3d):
    @functools.partial(
        pl.kernel,
        out_type=(jax.ShapeDtypeStruct((N_TILES, N_PAD), jnp.float32),
                  jax.ShapeDtypeStruct((N_TILES, N_PAD), jnp.float32)),
        mesh=_vector_mesh,
        scratch_types=[
            pltpu.VMEM((N_PAD,), jnp.float32),
            pltpu.VMEM((N_PAD,), jnp.float32),
            pltpu.VMEM((2, CHUNK, 2, GRAN), jnp.int32),
            pltpu.SemaphoreType.DMA((2,)),
        ],
        compiler_params=_sc_params,
    )
    def deg_kernel(e_hbm, out0_hbm, out1_hbm, acc0, acc1, ebuf, isem):
        cidx = lax.axis_index("c")
        sidx = lax.axis_index("s")
        wid = sidx * N_CORES + cidx

        zeros = jnp.zeros((LANES,), jnp.float32)
        ones = jnp.ones((LANES,), jnp.float32)

        @pl.loop(0, N_PAD // LANES)
        def _(i):
            acc0[pl.ds(i * LANES, LANES)] = zeros
            acc1[pl.ds(i * LANES, LANES)] = zeros

        def issue_idx(it, slot):
            c = wid + it * N_TILES

            @pl.when(c < N_CHUNKS)
            def _():
                pltpu.async_copy(e_hbm.at[pl.ds(c * CHUNK, CHUNK)],
                                 ebuf.at[slot], isem.at[slot])

        def drain_idx(it, slot):
            c = wid + it * N_TILES

            @pl.when(c < N_CHUNKS)
            def _():
                pltpu.make_async_copy(e_hbm.at[pl.ds(0, CHUNK)],
                                      ebuf.at[slot], isem.at[slot]).wait()

        def process(it, slot):
            c = wid + it * N_TILES

            @pl.when(c < N_CHUNKS)
            def _():
                @pl.loop(0, CHUNK)
                def _(j):
                    for k in range(GRAN // LANES):
                        i16s = ebuf[slot, j, 0, pl.ds(k * LANES, LANES)]
                        plsc.addupdate_scatter(acc0, [i16s], ones)
                        i16d = ebuf[slot, j, 1, pl.ds(k * LANES, LANES)]
                        plsc.addupdate_scatter(acc1, [i16d], ones)

        issue_idx(0, 0)

        @pl.loop(0, CHUNK_ITERS // 2)
        def _(h):
            it = h * 2
            drain_idx(it, 0)
            issue_idx(it + 1, 1)
            process(it, 0)
            drain_idx(it + 1, 1)
            issue_idx(it + 2, 0)
            process(it + 1, 1)

        pltpu.sync_copy(acc0, out0_hbm.at[wid])
        pltpu.sync_copy(acc1, out1_hbm.at[wid])

    return deg_kernel(e3d)


def _sc_propagate(h, e3d):
    @functools.partial(
        pl.kernel,
        out_type=jax.ShapeDtypeStruct((N_CORES, N_PAD, HID), jnp.float32),
        mesh=_vector_mesh,
        scratch_types=[
            pltpu.VMEM_SHARED((N_PAD, HID), jnp.float32),
            pltpu.VMEM((2, CHUNK, 2, GRAN), jnp.int32),
            pltpu.VMEM((2, CHUNK * GRAN, HID), jnp.float32),
            pltpu.VMEM((ZROWS, HID), jnp.float32),
            pltpu.SemaphoreType.DMA((2,)),
            pltpu.SemaphoreType.DMA((2,)),
            pltpu.SemaphoreType.DMA((2,)),
        ],
        compiler_params=_sc_params,
    )
    def prop_kernel(h_hbm, e_hbm, out_hbm, acc, ebuf, msgs, zbuf,
                    isem, gsem, ssem):
        cidx = lax.axis_index("c")
        sidx = lax.axis_index("s")
        wid = sidx * N_CORES + cidx

        zeros = jnp.zeros((LANES,), jnp.float32)

        @pl.loop(0, ZROWS)
        def _(i):
            zbuf[i, :] = zeros

        @pl.loop(0, ROWS_PER_SUB // ZROWS)
        def _(t):
            pltpu.sync_copy(zbuf, acc.at[pl.ds(sidx * ROWS_PER_SUB + t * ZROWS, ZROWS)])

        plsc.subcore_barrier()

        def issue_idx(it, slot):
            c = wid + it * N_TILES

            @pl.when(c < N_CHUNKS)
            def _():
                pltpu.async_copy(e_hbm.at[pl.ds(c * CHUNK, CHUNK)],
                                 ebuf.at[slot], isem.at[slot])

        def body(it, slot):
            c = wid + it * N_TILES
            cp = wid + (it - 1) * N_TILES

            @pl.when(c < N_CHUNKS)
            def _():
                pltpu.make_async_copy(e_hbm.at[pl.ds(0, CHUNK)],
                                      ebuf.at[slot], isem.at[slot]).wait()
                for j in range(CHUNK):
                    pltpu.async_copy(h_hbm.at[ebuf.at[slot, j, 0]],
                                     msgs.at[slot, pl.ds(j * GRAN, GRAN)],
                                     gsem.at[slot])

            @pl.when((it >= 1) & (cp < N_CHUNKS))
            def _():
                pltpu.make_async_copy(h_hbm.at[pl.ds(0, CHUNK * GRAN)],
                                      msgs.at[slot ^ 1],
                                      ssem.at[slot ^ 1]).wait()

            issue_idx(it + 1, slot ^ 1)

            @pl.when(c < N_CHUNKS)
            def _():
                pltpu.make_async_copy(h_hbm.at[pl.ds(0, CHUNK * GRAN)],
                                      msgs.at[slot], gsem.at[slot]).wait()
                for j in range(CHUNK):
                    pltpu.async_copy(msgs.at[slot, pl.ds(j * GRAN, GRAN)],
                                     acc.at[ebuf.at[slot, j, 1]],
                                     ssem.at[slot], add=True)

        issue_idx(0, 0)

        @pl.loop(0, CHUNK_ITERS // 2)
        def _(h2):
            it = h2 * 2
            body(it, 0)
            body(it + 1, 1)

        clast = wid + (CHUNK_ITERS - 1) * N_TILES

        @pl.when(clast < N_CHUNKS)
        def _():
            pltpu.make_async_copy(h_hbm.at[pl.ds(0, CHUNK * GRAN)],
                                  msgs.at[(CHUNK_ITERS - 1) % 2],
                                  ssem.at[(CHUNK_ITERS - 1) % 2]).wait()

        plsc.subcore_barrier()
        pltpu.sync_copy(
            acc.at[pl.ds(sidx * ROWS_PER_SUB, ROWS_PER_SUB)],
            out_hbm.at[cidx, pl.ds(sidx * ROWS_PER_SUB, ROWS_PER_SUB)],
        )

    return prop_kernel(h, e3d)


def _tc_norms(d0, d1, kexp):
    def expand(n, k):
        e = lax.dot_general(n, k, (((1,), (0,)), ((), ())),
                            preferred_element_type=jnp.float32)
        return e.reshape(NB, LANES, 128).reshape(PK_ROWS, 128)

    def body(d0_ref, d1_ref, k_ref, ne0_ref, ne1_ref):
        s0 = jnp.sum(d0_ref[...].reshape(N_TILES, NB, 128), axis=0)
        s1 = jnp.sum(d1_ref[...].reshape(N_TILES, NB, 128), axis=0)
        k = k_ref[...]
        ne0_ref[...] = expand(lax.rsqrt(jnp.maximum(s0, 1.0)), k)
        ne1_ref[...] = expand(lax.rsqrt(jnp.maximum(s1, 1.0)), k)

    return pl.pallas_call(
        body,
        out_shape=(jax.ShapeDtypeStruct((PK_ROWS, 128), jnp.float32),
                   jax.ShapeDtypeStruct((PK_ROWS, 128), jnp.float32)),
    )(d0, d1, kexp)


def _tc_matmul1(xt, w1):
    cb = 2048
    grid = -(-N_NODES // cb)

    def body(xt_ref, w_ref, o_ref):
        xb = xt_ref[...].astype(jnp.bfloat16)
        wb = w_ref[...].astype(jnp.bfloat16)
        prod = lax.dot_general(xb, wb, (((0,), (0,)), ((), ())),
                               preferred_element_type=jnp.float32)
        pr = prod.reshape(cb // 8, 8, HID)
        o_ref[...] = jnp.concatenate([pr[:, m, :] for m in range(8)], axis=1)

    return pl.pallas_call(
        body,
        grid=(grid,),
        in_specs=[
            pl.BlockSpec((IN_FEATS, cb), lambda i: (0, i)),
            pl.BlockSpec((IN_FEATS, HID), lambda i: (0, 0)),
        ],
        out_specs=pl.BlockSpec((cb // 8, 128), lambda i: (i, 0)),
        out_shape=jax.ShapeDtypeStruct((PK_ROWS, 128), jnp.float32),
    )(xt, w1)


def _tc_scale(xwp, ne0):
    def body(xw_ref, n_ref, o_ref):
        o_ref[...] = xw_ref[...] * n_ref[...]

    return pl.pallas_call(
        body,
        out_shape=jax.ShapeDtypeStruct((PK_ROWS, 128), jnp.float32),
    )(xwp, ne0)


def _tc_mid(p, ne0, ne1, b1e):
    def body(p_ref, n0_ref, n1_ref, b_ref, o_ref):
        pv = p_ref[...].reshape(2, PK_ROWS, 128)
        agg = (pv[0] + pv[1]) * n1_ref[...] + b_ref[...]
        o_ref[...] = jnp.maximum(agg, 0.0) * n0_ref[...]

    return pl.pallas_call(
        body,
        out_shape=jax.ShapeDtypeStruct((PK_ROWS, 128), jnp.float32),
    )(p, ne0, ne1, b1e)


def _tc_final(p, ne1, w2, b2):
    rb = 368
    grid = PK_ROWS // rb

    def body(p0_ref, p1_ref, n_ref, w_ref, b_ref, o_ref):
        agg = (p0_ref[...] + p1_ref[...]) * n_ref[...]
        un = jnp.stack([agg[:, m * HID:(m + 1) * HID] for m in range(8)],
                       axis=1).reshape(8 * rb, HID)
        res = lax.dot_general(w_ref[...], un, (((0,), (1,)), ((), ())),
                              preferred_element_type=jnp.float32)
        o_ref[...] = res + b_ref[...].reshape(OUT, 1)

    return pl.pallas_call(
        body,
        grid=(grid,),
        in_specs=[
            pl.BlockSpec((rb, 128), lambda i: (i, 0)),
            pl.BlockSpec((rb, 128), lambda i: (i + PK_ROWS // rb, 0)),
            pl.BlockSpec((rb, 128), lambda i: (i, 0)),
            pl.BlockSpec((HID, OUT), lambda i: (0, 0)),
            pl.BlockSpec((1, OUT), lambda i: (0, 0)),
        ],
        out_specs=pl.BlockSpec((OUT, 8 * rb), lambda i: (0, i)),
        out_shape=jax.ShapeDtypeStruct((OUT, N_NODES), jnp.float32),
    )(p, p, ne1, w2, b2.reshape(1, OUT))


def kernel(features_, edge_index, W1, b1, W2, b2):
    ei = edge_index.astype(jnp.int32)
    e3d = ei.reshape(2, N_GRAN, GRAN).transpose(1, 0, 2)

    b1e = jnp.tile(b1, 8).reshape(1, 128)
    ll = jnp.arange(LANES * 128)[None, :]
    kexp = (jnp.arange(128)[:, None] == 8 * (ll // 128) + (ll % 128) // LANES
            ).astype(jnp.float32)

    deg0, deg1 = _sc_degrees(e3d)
    ne0, ne1 = _tc_norms(deg0.reshape(DG_ROWS, 128),
                         deg1.reshape(DG_ROWS, 128), kexp)
    xw1p = _tc_matmul1(features_.T, W1)
    h1sp = _tc_scale(xw1p, ne0)
    p1 = _sc_propagate(h1sp.reshape(N_PAD, HID), e3d)
    yp = _tc_mid(p1.reshape(DG_ROWS, 128), ne0, ne1, b1e)
    p2 = _sc_propagate(yp.reshape(N_PAD, HID), e3d)
    return _tc_final(p2.reshape(DG_ROWS, 128), ne1, W2, b2).T
